# Initial kernel scaffold; baseline (speedup 1.0000x reference)
import jax
import jax.numpy as jnp
from jax.experimental import pallas as pl


def kernel(x, edge_index, batch, params):
    raise NotImplementedError("write your pallas kernel here")


# speedup vs baseline: 4.7375x; 4.7375x over previous
import functools

import jax
import jax.numpy as jnp
from jax import lax
from jax.experimental import pallas as pl
from jax.experimental.pallas import tpu as pltpu
from jax.experimental.pallas import tpu_sc as plsc

NC, NS = 2, 16
NW = NC * NS
G = 128
R = 10240
E_PAD = 163840
CB_E, CH_E = 40, 128

LAYERS = (dict(N=10000, H=32, K=8000),
          dict(N=8000, H=64, K=6400),
          dict(N=6400, H=128, K=5120))


def _make_sc_msg(D):
    mesh = plsc.VectorSubcoreMesh(core_axis_name="c", subcore_axis_name="s")
    rpt = R // NS

    @functools.partial(
        pl.kernel, mesh=mesh,
        out_type=(jax.ShapeDtypeStruct((R, D), jnp.float32),
                  jax.ShapeDtypeStruct((R, D), jnp.float32)),
        scratch_types=[
            pltpu.VMEM_SHARED((R, D), jnp.float32),
            pltpu.VMEM((CB_E, CH_E), jnp.int32),
            pltpu.VMEM((CB_E, CH_E), jnp.int32),
            pltpu.VMEM((CH_E, D), jnp.float32),
            pltpu.SemaphoreType.DMA,
        ],
    )
    def k(x_hbm, src_hbm, dst_hbm, zero_hbm, out0, out1,
          acc, src_v, dst_v, rows_v, sem):
        cid = lax.axis_index("c")
        sid = lax.axis_index("s")
        wid = cid * NS + sid
        pltpu.sync_copy(zero_hbm.at[pl.ds(sid * rpt, rpt)],
                        acc.at[pl.ds(sid * rpt, rpt)])
        plsc.subcore_barrier()
        pltpu.sync_copy(src_hbm.at[wid], src_v)
        pltpu.sync_copy(dst_hbm.at[wid], dst_v)

        def body(j, carry):
            pltpu.async_copy(x_hbm.at[src_v.at[j]], rows_v, sem).wait()
            pltpu.sync_copy(rows_v, acc.at[dst_v.at[j]], add=True)
            return carry

        lax.fori_loop(0, CB_E, body, 0)
        plsc.subcore_barrier()

        @pl.when(cid == 0)
        def _():
            pltpu.sync_copy(acc.at[pl.ds(sid * rpt, rpt)],
                            out0.at[pl.ds(sid * rpt, rpt)])

        @pl.when(cid == 1)
        def _():
            pltpu.sync_copy(acc.at[pl.ds(sid * rpt, rpt)],
                            out1.at[pl.ds(sid * rpt, rpt)])

    return k


def _sc_msg(D, x, src3, dst3):
    z = jnp.zeros((R, D), jnp.float32)
    return _make_sc_msg(D)(x, src3, dst3, z)


def _tc_conv_body(n_kept, H, xin, agg0, agg1, keep, w1, b1, g1, be1,
                  w2, b2, g2, be2, wp, bp, hs_out, sc_out):
    kp = keep[...]
    z = (xin[...] + agg0[...] + agg1[...]) * kp
    n = jnp.float32(n_kept)

    def bnrelu(h, gam, bet):
        h = h * kp
        m = jnp.sum(h, axis=0, keepdims=True) / n
        ex2 = jnp.sum(h * h, axis=0, keepdims=True) / n
        v = ex2 - m * m
        h = (h - m) * lax.rsqrt(v + 1e-5) * gam[...] + bet[...]
        return jnp.maximum(h, 0.0) * kp

    h = bnrelu(jnp.dot(z, w1[...], preferred_element_type=jnp.float32)
               + b1[...], g1, be1)
    h = bnrelu(jnp.dot(h, w2[...], preferred_element_type=jnp.float32)
               + b2[...], g2, be2)
    score = jax.nn.sigmoid(
        jnp.dot(h, wp[...], preferred_element_type=jnp.float32) + bp[...])
    sc_out[...] = jnp.where(kp > 0, score, -1.0)
    hs_out[:, :H] = h * score * kp
    if H < 128:
        hs_out[:, H:] = jnp.zeros((z.shape[0], 128 - H), jnp.float32)


def _tc_conv(n_kept, xin, agg0, agg1, keep, p, pp):
    H = p["lin1"]["W"].shape[1]
    w1 = p["lin1"]["W"]
    w1 = jnp.zeros((128, H), jnp.float32).at[:w1.shape[0]].set(w1)
    return pl.pallas_call(
        functools.partial(_tc_conv_body, n_kept, H),
        out_shape=(jax.ShapeDtypeStruct((R, 128), jnp.float32),
                   jax.ShapeDtypeStruct((R, 1), jnp.float32)),
    )(xin, agg0, agg1, keep,
      w1, p["lin1"]["b"].reshape(1, -1),
      p["bn1"]["gamma"].reshape(1, -1), p["bn1"]["beta"].reshape(1, -1),
      p["lin2"]["W"], p["lin2"]["b"].reshape(1, -1),
      p["bn2"]["gamma"].reshape(1, -1), p["bn2"]["beta"].reshape(1, -1),
      pp["W"], pp["b"].reshape(1, 1))


def _tc_pool_body(kk, s_ref, keep_ref):
    s = s_ref[...]
    Rr = s.shape[0]
    idx = (lax.broadcasted_iota(jnp.int32, (Rr, 256), 0) * 256
           + lax.broadcasted_iota(jnp.int32, (Rr, 256), 1))

    def cnt_gt(t):
        return jnp.sum((s > t).astype(jnp.int32))

    def bs_val(i, lh):
        lo, hi = lh
        mid = (lo + hi) // 2
        c = cnt_gt(lax.bitcast_convert_type(mid, jnp.float32))
        return (jnp.where(c >= kk, mid, lo), jnp.where(c >= kk, hi, mid))

    lo, hi = lax.fori_loop(0, 31, bs_val,
                           (jnp.int32(0), jnp.int32(0x3F800000)))
    vk = lax.bitcast_convert_type(hi, jnp.float32)
    m = kk - cnt_gt(vk)
    ties = (s == vk)

    def bs_idx(i, lh):
        lo2, hi2 = lh
        mid = (lo2 + hi2) // 2
        c = jnp.sum((ties & (idx < mid)).astype(jnp.int32))
        return (jnp.where(c >= m, lo2, mid), jnp.where(c >= m, mid, hi2))

    lo2, hi2 = lax.fori_loop(0, 15, bs_idx,
                             (jnp.int32(0), jnp.int32(Rr * 256)))
    keep_ref[...] = ((s > vk) | (ties & (idx < hi2))).astype(jnp.float32)


def _tc_pool(kk, score):
    return pl.pallas_call(
        functools.partial(_tc_pool_body, kk),
        out_shape=jax.ShapeDtypeStruct((R // 256, 256), jnp.float32),
    )(score.reshape(R // 256, 256))


def _tc_readout_body(D, hs_ref, keep_ref, brow_ref, bcol_ref, out_ref,
                     xn_ref):
    keep = keep_ref[...]
    xn_ref[...] = hs_ref[...] * keep
    xc = hs_ref[:, :D] * keep
    brow = brow_ref[...]
    bcol = bcol_ref[...]
    onehot = ((lax.broadcasted_iota(jnp.int32, (G, R), 0)
               == jnp.broadcast_to(brow, (G, R))).astype(jnp.float32)
              * jnp.broadcast_to(keep.reshape(1, R), (G, R)))
    seg = jnp.dot(onehot, xc, preferred_element_type=jnp.float32)
    cnt = jnp.sum(onehot, axis=1, keepdims=True)
    out_ref[:, D:] = seg / jnp.maximum(cnt, 1.0)
    kcol = keep > 0

    def max_row(g):
        mrow = jnp.max(jnp.where((bcol == g) & kcol, xc, -3e38), axis=0,
                       keepdims=True)
        return jnp.maximum(mrow, 0.0)

    for gb in range(G // 8):
        out_ref[8 * gb:8 * gb + 8, :D] = jnp.concatenate(
            [max_row(8 * gb + j) for j in range(8)], axis=0)


def _tc_readout(D, hs, keep, brow, bcol):
    return pl.pallas_call(
        functools.partial(_tc_readout_body, D),
        out_shape=(jax.ShapeDtypeStruct((G, 2 * D), jnp.float32),
                   jax.ShapeDtypeStruct((R, 128), jnp.float32)),
    )(hs, keep, brow, bcol)


def _tc_dec_body(x1_ref, x2_ref, x3_ref,
                 w31, b31, g31, be31, w32, b32, g32, be32,
                 w21, b21, g21, be21, w22, b22, g22, be22,
                 w1, bb1, out_ref):
    def bnrelu(h, gam, bet):
        m = jnp.mean(h, axis=0, keepdims=True)
        v = jnp.mean(h * h, axis=0, keepdims=True) - m * m
        h = (h - m) * lax.rsqrt(v + 1e-5) * gam[...] + bet[...]
        return jnp.maximum(h, 0.0)

    def mlp(x, wA, bA, gA, beA, wB, bB, gB, beB):
        h = bnrelu(jnp.dot(x, wA[...], preferred_element_type=jnp.float32)
                   + bA[...], gA, beA)
        return bnrelu(jnp.dot(h, wB[...], preferred_element_type=jnp.float32)
                      + bB[...], gB, beB)

    xd3 = mlp(x3_ref[...], w31, b31, g31, be31, w32, b32, g32, be32)
    xd2 = mlp(xd3 + x2_ref[...], w21, b21, g21, be21, w22, b22, g22, be22)
    logits = (jnp.dot(xd2 + x1_ref[...], w1[...],
                      preferred_element_type=jnp.float32) + bb1[...])
    mx = jnp.max(logits, axis=1, keepdims=True)
    lse = jnp.log(jnp.sum(jnp.exp(logits - mx), axis=1, keepdims=True)) + mx
    out_ref[...] = logits - lse


def _tc_decoder(x1, x2, x3, d3, d2, d1):
    def flat(p):
        return (p["lin1"]["W"], p["lin1"]["b"].reshape(1, -1),
                p["bn1"]["gamma"].reshape(1, -1),
                p["bn1"]["beta"].reshape(1, -1),
                p["lin2"]["W"], p["lin2"]["b"].reshape(1, -1),
                p["bn2"]["gamma"].reshape(1, -1),
                p["bn2"]["beta"].reshape(1, -1))

    return pl.pallas_call(
        _tc_dec_body,
        out_shape=jax.ShapeDtypeStruct((G, d1["W"].shape[1]), jnp.float32),
    )(x1, x2, x3, *flat(d3), *flat(d2), d1["W"], d1["b"].reshape(1, -1))


def _layer(cfg, x, src3, dst3, keep, brow, bcol, conv_p, pool_p):
    agg0, agg1 = _sc_msg(128, x, src3, dst3)
    hs, score = _tc_conv(cfg["N"], x, agg0, agg1, keep, conv_p, pool_p)
    keep_new = _tc_pool(cfg["K"], score).reshape(R, 1)
    xr, xnext = _tc_readout(cfg["H"], hs, keep_new, brow, bcol)
    return xnext, keep_new, xr


def kernel(x, edge_index, batch, params):
    n = x.shape[0]
    xp = jnp.zeros((R, x.shape[1]), jnp.float32).at[:n].set(x)
    n_e = edge_index.shape[1]
    src3 = (jnp.zeros((E_PAD,), jnp.int32).at[:n_e].set(edge_index[0])
            .reshape(NW, CB_E, CH_E))
    dst3 = (jnp.full((E_PAD,), R - 1, jnp.int32).at[:n_e].set(edge_index[1])
            .reshape(NW, CB_E, CH_E))
    bp = jnp.full((R,), -1, jnp.int32).at[:n].set(batch)
    brow = bp.reshape(1, R)
    bcol = bp.reshape(R, 1)
    keep = (jnp.arange(R, dtype=jnp.int32) < n).astype(jnp.float32)
    keep = keep.reshape(R, 1)

    h, keep, x1 = _layer(LAYERS[0], xp, src3, dst3, keep, brow, bcol,
                         params["conv1"], params["pool1"])
    h, keep, x2 = _layer(LAYERS[1], h, src3, dst3, keep, brow, bcol,
                         params["conv2"], params["pool2"])
    h, keep, x3 = _layer(LAYERS[2], h, src3, dst3, keep, brow, bcol,
                         params["conv3"], params["pool3"])
    return _tc_decoder(x1, x2, x3, params["dec3"], params["dec2"],
                       params["dec1"])


# speedup vs baseline: 4.9438x; 1.0435x over previous
import functools

import jax
import jax.numpy as jnp
from jax import lax
from jax.experimental import pallas as pl
from jax.experimental.pallas import tpu as pltpu
from jax.experimental.pallas import tpu_sc as plsc

NC, NS = 2, 16
NW = NC * NS
G = 128
R = 10240
E_PAD = 163840
CB_E, CH_E = 40, 128

LAYERS = (dict(N=10000, H=32, K=8000),
          dict(N=8000, H=64, K=6400),
          dict(N=6400, H=128, K=5120))


def _make_sc_msg(D):
    mesh = plsc.VectorSubcoreMesh(core_axis_name="c", subcore_axis_name="s")
    rpt = R // NS

    @functools.partial(
        pl.kernel, mesh=mesh,
        out_type=(jax.ShapeDtypeStruct((R, D), jnp.float32),
                  jax.ShapeDtypeStruct((R, D), jnp.float32)),
        scratch_types=[
            pltpu.VMEM_SHARED((R, D), jnp.float32),
            pltpu.VMEM((CB_E, CH_E), jnp.int32),
            pltpu.VMEM((CB_E, CH_E), jnp.int32),
            pltpu.VMEM((CH_E, D), jnp.float32),
            pltpu.VMEM((CH_E, D), jnp.float32),
            pltpu.SemaphoreType.DMA,
            pltpu.SemaphoreType.DMA,
        ],
    )
    def k(x_hbm, src_hbm, dst_hbm, zero_hbm, out0, out1,
          acc, src_v, dst_v, rows0_v, rows1_v, sem0, sem1):
        cid = lax.axis_index("c")
        sid = lax.axis_index("s")
        wid = cid * NS + sid
        pltpu.sync_copy(zero_hbm.at[pl.ds(sid * rpt, rpt)],
                        acc.at[pl.ds(sid * rpt, rpt)])
        plsc.subcore_barrier()
        pltpu.sync_copy(src_hbm.at[wid], src_v)
        pltpu.sync_copy(dst_hbm.at[wid], dst_v)

        pltpu.async_copy(x_hbm.at[src_v.at[0]], rows0_v, sem0)

        def stage(j, rows_v, sem, rows_nxt, sem_nxt):
            pltpu.make_async_copy(x_hbm, rows_v, sem).wait()

            @pl.when(j < CB_E - 1)
            def _():
                pltpu.async_copy(x_hbm.at[src_v.at[j + 1]], rows_nxt,
                                 sem_nxt)

            pltpu.sync_copy(rows_v, acc.at[dst_v.at[j]], add=True)

        def body(j, carry):
            @pl.when(j % 2 == 0)
            def _():
                stage(j, rows0_v, sem0, rows1_v, sem1)

            @pl.when(j % 2 == 1)
            def _():
                stage(j, rows1_v, sem1, rows0_v, sem0)

            return carry

        lax.fori_loop(0, CB_E, body, 0)
        plsc.subcore_barrier()

        @pl.when(cid == 0)
        def _():
            pltpu.sync_copy(acc.at[pl.ds(sid * rpt, rpt)],
                            out0.at[pl.ds(sid * rpt, rpt)])

        @pl.when(cid == 1)
        def _():
            pltpu.sync_copy(acc.at[pl.ds(sid * rpt, rpt)],
                            out1.at[pl.ds(sid * rpt, rpt)])

    return k


def _sc_msg(D, x, src3, dst3):
    z = jnp.zeros((R, D), jnp.float32)
    return _make_sc_msg(D)(x, src3, dst3, z)


def _tc_conv_body(n_kept, H, xin, agg0, agg1, keep, w1, b1, g1, be1,
                  w2, b2, g2, be2, wp, bp, hs_out, sc_out):
    kp = keep[...]
    z = (xin[...] + agg0[...] + agg1[...]) * kp
    n = jnp.float32(n_kept)

    def bnrelu(h, gam, bet):
        h = h * kp
        m = jnp.sum(h, axis=0, keepdims=True) / n
        ex2 = jnp.sum(h * h, axis=0, keepdims=True) / n
        v = ex2 - m * m
        h = (h - m) * lax.rsqrt(v + 1e-5) * gam[...] + bet[...]
        return jnp.maximum(h, 0.0) * kp

    h = bnrelu(jnp.dot(z, w1[...], preferred_element_type=jnp.float32)
               + b1[...], g1, be1)
    h = bnrelu(jnp.dot(h, w2[...], preferred_element_type=jnp.float32)
               + b2[...], g2, be2)
    score = jax.nn.sigmoid(
        jnp.dot(h, wp[...], preferred_element_type=jnp.float32) + bp[...])
    sc_out[...] = jnp.where(kp > 0, score, -1.0)
    hs_out[:, :H] = h * score * kp
    if H < 128:
        hs_out[:, H:] = jnp.zeros((z.shape[0], 128 - H), jnp.float32)


def _tc_conv(n_kept, xin, agg0, agg1, keep, p, pp):
    H = p["lin1"]["W"].shape[1]
    w1 = p["lin1"]["W"]
    w1 = jnp.zeros((128, H), jnp.float32).at[:w1.shape[0]].set(w1)
    return pl.pallas_call(
        functools.partial(_tc_conv_body, n_kept, H),
        out_shape=(jax.ShapeDtypeStruct((R, 128), jnp.float32),
                   jax.ShapeDtypeStruct((R, 1), jnp.float32)),
    )(xin, agg0, agg1, keep,
      w1, p["lin1"]["b"].reshape(1, -1),
      p["bn1"]["gamma"].reshape(1, -1), p["bn1"]["beta"].reshape(1, -1),
      p["lin2"]["W"], p["lin2"]["b"].reshape(1, -1),
      p["bn2"]["gamma"].reshape(1, -1), p["bn2"]["beta"].reshape(1, -1),
      pp["W"], pp["b"].reshape(1, 1))


def _tc_pool_body(kk, s_ref, keep_ref):
    s = s_ref[...]
    Rr = s.shape[0]
    idx = (lax.broadcasted_iota(jnp.int32, (Rr, 256), 0) * 256
           + lax.broadcasted_iota(jnp.int32, (Rr, 256), 1))

    def cnt_gt(t):
        return jnp.sum((s > t).astype(jnp.int32))

    def bs_val(i, lh):
        lo, hi = lh
        mid = (lo + hi) // 2
        c = cnt_gt(lax.bitcast_convert_type(mid, jnp.float32))
        return (jnp.where(c >= kk, mid, lo), jnp.where(c >= kk, hi, mid))

    lo, hi = lax.fori_loop(0, 31, bs_val,
                           (jnp.int32(0), jnp.int32(0x3F800000)))
    vk = lax.bitcast_convert_type(hi, jnp.float32)
    m = kk - cnt_gt(vk)
    ties = (s == vk)

    def bs_idx(i, lh):
        lo2, hi2 = lh
        mid = (lo2 + hi2) // 2
        c = jnp.sum((ties & (idx < mid)).astype(jnp.int32))
        return (jnp.where(c >= m, lo2, mid), jnp.where(c >= m, mid, hi2))

    lo2, hi2 = lax.fori_loop(0, 15, bs_idx,
                             (jnp.int32(0), jnp.int32(Rr * 256)))
    keep_ref[...] = ((s > vk) | (ties & (idx < hi2))).astype(jnp.float32)


def _tc_pool(kk, score):
    return pl.pallas_call(
        functools.partial(_tc_pool_body, kk),
        out_shape=jax.ShapeDtypeStruct((R // 256, 256), jnp.float32),
    )(score.reshape(R // 256, 256))


def _tc_readout_body(D, hs_ref, keep_ref, brow_ref, bcol_ref, out_ref,
                     xn_ref):
    keep = keep_ref[...]
    xn_ref[...] = hs_ref[...] * keep
    xc = hs_ref[:, :D] * keep
    brow = brow_ref[...]
    bcol = bcol_ref[...]
    onehot = ((lax.broadcasted_iota(jnp.int32, (G, R), 0)
               == jnp.broadcast_to(brow, (G, R))).astype(jnp.float32)
              * jnp.broadcast_to(keep.reshape(1, R), (G, R)))
    seg = jnp.dot(onehot, xc, preferred_element_type=jnp.float32)
    cnt = jnp.sum(onehot, axis=1, keepdims=True)
    out_ref[:, D:] = seg / jnp.maximum(cnt, 1.0)
    kcol = keep > 0

    def max_row(g):
        mrow = jnp.max(jnp.where((bcol == g) & kcol, xc, -3e38), axis=0,
                       keepdims=True)
        return jnp.maximum(mrow, 0.0)

    for gb in range(G // 8):
        out_ref[8 * gb:8 * gb + 8, :D] = jnp.concatenate(
            [max_row(8 * gb + j) for j in range(8)], axis=0)


def _tc_readout(D, hs, keep, brow, bcol):
    return pl.pallas_call(
        functools.partial(_tc_readout_body, D),
        out_shape=(jax.ShapeDtypeStruct((G, 2 * D), jnp.float32),
                   jax.ShapeDtypeStruct((R, 128), jnp.float32)),
    )(hs, keep, brow, bcol)


def _tc_dec_body(x1_ref, x2_ref, x3_ref,
                 w31, b31, g31, be31, w32, b32, g32, be32,
                 w21, b21, g21, be21, w22, b22, g22, be22,
                 w1, bb1, out_ref):
    def bnrelu(h, gam, bet):
        m = jnp.mean(h, axis=0, keepdims=True)
        v = jnp.mean(h * h, axis=0, keepdims=True) - m * m
        h = (h - m) * lax.rsqrt(v + 1e-5) * gam[...] + bet[...]
        return jnp.maximum(h, 0.0)

    def mlp(x, wA, bA, gA, beA, wB, bB, gB, beB):
        h = bnrelu(jnp.dot(x, wA[...], preferred_element_type=jnp.float32)
                   + bA[...], gA, beA)
        return bnrelu(jnp.dot(h, wB[...], preferred_element_type=jnp.float32)
                      + bB[...], gB, beB)

    xd3 = mlp(x3_ref[...], w31, b31, g31, be31, w32, b32, g32, be32)
    xd2 = mlp(xd3 + x2_ref[...], w21, b21, g21, be21, w22, b22, g22, be22)
    logits = (jnp.dot(xd2 + x1_ref[...], w1[...],
                      preferred_element_type=jnp.float32) + bb1[...])
    mx = jnp.max(logits, axis=1, keepdims=True)
    lse = jnp.log(jnp.sum(jnp.exp(logits - mx), axis=1, keepdims=True)) + mx
    out_ref[...] = logits - lse


def _tc_decoder(x1, x2, x3, d3, d2, d1):
    def flat(p):
        return (p["lin1"]["W"], p["lin1"]["b"].reshape(1, -1),
                p["bn1"]["gamma"].reshape(1, -1),
                p["bn1"]["beta"].reshape(1, -1),
                p["lin2"]["W"], p["lin2"]["b"].reshape(1, -1),
                p["bn2"]["gamma"].reshape(1, -1),
                p["bn2"]["beta"].reshape(1, -1))

    return pl.pallas_call(
        _tc_dec_body,
        out_shape=jax.ShapeDtypeStruct((G, d1["W"].shape[1]), jnp.float32),
    )(x1, x2, x3, *flat(d3), *flat(d2), d1["W"], d1["b"].reshape(1, -1))


def _layer(cfg, x, src3, dst3, keep, brow, bcol, conv_p, pool_p):
    agg0, agg1 = _sc_msg(128, x, src3, dst3)
    hs, score = _tc_conv(cfg["N"], x, agg0, agg1, keep, conv_p, pool_p)
    keep_new = _tc_pool(cfg["K"], score).reshape(R, 1)
    xr, xnext = _tc_readout(cfg["H"], hs, keep_new, brow, bcol)
    return xnext, keep_new, xr


def kernel(x, edge_index, batch, params):
    n = x.shape[0]
    xp = jnp.zeros((R, x.shape[1]), jnp.float32).at[:n].set(x)
    n_e = edge_index.shape[1]
    src3 = (jnp.zeros((E_PAD,), jnp.int32).at[:n_e].set(edge_index[0])
            .reshape(NW, CB_E, CH_E))
    dst3 = (jnp.full((E_PAD,), R - 1, jnp.int32).at[:n_e].set(edge_index[1])
            .reshape(NW, CB_E, CH_E))
    bp = jnp.full((R,), -1, jnp.int32).at[:n].set(batch)
    brow = bp.reshape(1, R)
    bcol = bp.reshape(R, 1)
    keep = (jnp.arange(R, dtype=jnp.int32) < n).astype(jnp.float32)
    keep = keep.reshape(R, 1)

    h, keep, x1 = _layer(LAYERS[0], xp, src3, dst3, keep, brow, bcol,
                         params["conv1"], params["pool1"])
    h, keep, x2 = _layer(LAYERS[1], h, src3, dst3, keep, brow, bcol,
                         params["conv2"], params["pool2"])
    h, keep, x3 = _layer(LAYERS[2], h, src3, dst3, keep, brow, bcol,
                         params["conv3"], params["pool3"])
    return _tc_decoder(x1, x2, x3, params["dec3"], params["dec2"],
                       params["dec1"])


# speedup vs baseline: 5.4991x; 1.1123x over previous
import functools

import jax
import jax.numpy as jnp
from jax import lax
from jax.experimental import pallas as pl
from jax.experimental.pallas import tpu as pltpu
from jax.experimental.pallas import tpu_sc as plsc

NC, NS = 2, 16
NW = NC * NS
G = 128
R = 10240
E_PAD = 163840
CB_E, CH_E = 40, 128

LAYERS = (dict(N=10000, H=32, K=8000),
          dict(N=8000, H=64, K=6400),
          dict(N=6400, H=128, K=5120))


def _make_sc_msg(D):
    mesh = plsc.VectorSubcoreMesh(core_axis_name="c", subcore_axis_name="s")
    rpt = R // NS

    @functools.partial(
        pl.kernel, mesh=mesh,
        out_type=(jax.ShapeDtypeStruct((R, D), jnp.float32),
                  jax.ShapeDtypeStruct((R, D), jnp.float32)),
        scratch_types=[
            pltpu.VMEM_SHARED((R, D), jnp.float32),
            pltpu.VMEM((CB_E, CH_E), jnp.int32),
            pltpu.VMEM((CB_E, CH_E), jnp.int32),
            pltpu.VMEM((CH_E, D), jnp.float32),
            pltpu.VMEM((CH_E, D), jnp.float32),
            pltpu.SemaphoreType.DMA,
            pltpu.SemaphoreType.DMA,
        ],
    )
    def k(x_hbm, src_hbm, dst_hbm, zero_hbm, out0, out1,
          acc, src_v, dst_v, rows0_v, rows1_v, sem0, sem1):
        cid = lax.axis_index("c")
        sid = lax.axis_index("s")
        wid = cid * NS + sid
        pltpu.sync_copy(zero_hbm.at[pl.ds(sid * rpt, rpt)],
                        acc.at[pl.ds(sid * rpt, rpt)])
        plsc.subcore_barrier()
        pltpu.sync_copy(src_hbm.at[wid], src_v)
        pltpu.sync_copy(dst_hbm.at[wid], dst_v)

        pltpu.async_copy(x_hbm.at[src_v.at[0]], rows0_v, sem0)

        def stage(j, rows_v, sem, rows_nxt, sem_nxt):
            pltpu.make_async_copy(x_hbm, rows_v, sem).wait()

            @pl.when(j < CB_E - 1)
            def _():
                pltpu.async_copy(x_hbm.at[src_v.at[j + 1]], rows_nxt,
                                 sem_nxt)

            pltpu.sync_copy(rows_v, acc.at[dst_v.at[j]], add=True)

        def body(j, carry):
            @pl.when(j % 2 == 0)
            def _():
                stage(j, rows0_v, sem0, rows1_v, sem1)

            @pl.when(j % 2 == 1)
            def _():
                stage(j, rows1_v, sem1, rows0_v, sem0)

            return carry

        lax.fori_loop(0, CB_E, body, 0)
        plsc.subcore_barrier()

        @pl.when(cid == 0)
        def _():
            pltpu.sync_copy(acc.at[pl.ds(sid * rpt, rpt)],
                            out0.at[pl.ds(sid * rpt, rpt)])

        @pl.when(cid == 1)
        def _():
            pltpu.sync_copy(acc.at[pl.ds(sid * rpt, rpt)],
                            out1.at[pl.ds(sid * rpt, rpt)])

    return k


def _sc_msg(D, x, src3, dst3):
    z = jnp.zeros((R, D), jnp.float32)
    return _make_sc_msg(D)(x, src3, dst3, z)


def _tc_conv_body(n_kept, H, xin, agg0, agg1, keep, w1, b1, g1, be1,
                  w2, b2, g2, be2, wp, bp, hs_out, sc_out):
    kp = keep[...]
    z = (xin[...] + agg0[...] + agg1[...]) * kp
    n = jnp.float32(n_kept)

    def bnrelu(h, gam, bet):
        h = h * kp
        m = jnp.sum(h, axis=0, keepdims=True) / n
        ex2 = jnp.sum(h * h, axis=0, keepdims=True) / n
        v = ex2 - m * m
        h = (h - m) * lax.rsqrt(v + 1e-5) * gam[...] + bet[...]
        return jnp.maximum(h, 0.0) * kp

    h = bnrelu(jnp.dot(z, w1[...], preferred_element_type=jnp.float32)
               + b1[...], g1, be1)
    h = bnrelu(jnp.dot(h, w2[...], preferred_element_type=jnp.float32)
               + b2[...], g2, be2)
    score = jax.nn.sigmoid(
        jnp.dot(h, wp[...], preferred_element_type=jnp.float32) + bp[...])
    sc_out[...] = jnp.where(kp > 0, score, -1.0)
    hs_out[:, :H] = h * score * kp
    if H < 128:
        hs_out[:, H:] = jnp.zeros((z.shape[0], 128 - H), jnp.float32)


def _tc_conv(n_kept, xin, agg0, agg1, keep, p, pp):
    H = p["lin1"]["W"].shape[1]
    w1 = p["lin1"]["W"]
    w1 = jnp.zeros((128, H), jnp.float32).at[:w1.shape[0]].set(w1)
    return pl.pallas_call(
        functools.partial(_tc_conv_body, n_kept, H),
        out_shape=(jax.ShapeDtypeStruct((R, 128), jnp.float32),
                   jax.ShapeDtypeStruct((R, 1), jnp.float32)),
    )(xin, agg0, agg1, keep,
      w1, p["lin1"]["b"].reshape(1, -1),
      p["bn1"]["gamma"].reshape(1, -1), p["bn1"]["beta"].reshape(1, -1),
      p["lin2"]["W"], p["lin2"]["b"].reshape(1, -1),
      p["bn2"]["gamma"].reshape(1, -1), p["bn2"]["beta"].reshape(1, -1),
      pp["W"], pp["b"].reshape(1, 1))


def _tc_pool_body(kk, s_ref, keep_ref):
    s = s_ref[...]
    Rr = s.shape[0]
    idx = (lax.broadcasted_iota(jnp.int32, (Rr, 256), 0) * 256
           + lax.broadcasted_iota(jnp.int32, (Rr, 256), 1))

    def cnt_gt(t):
        return jnp.sum((s > t).astype(jnp.int32))

    def bs_val(i, lh):
        lo, hi = lh
        mid = (lo + hi) // 2
        c = cnt_gt(lax.bitcast_convert_type(mid, jnp.float32))
        return (jnp.where(c >= kk, mid, lo), jnp.where(c >= kk, hi, mid))

    lo, hi = lax.fori_loop(0, 31, bs_val,
                           (jnp.int32(0), jnp.int32(0x3F800000)))
    vk = lax.bitcast_convert_type(hi, jnp.float32)
    m = kk - cnt_gt(vk)
    ties = (s == vk)

    def bs_idx(i, lh):
        lo2, hi2 = lh
        mid = (lo2 + hi2) // 2
        c = jnp.sum((ties & (idx < mid)).astype(jnp.int32))
        return (jnp.where(c >= m, lo2, mid), jnp.where(c >= m, mid, hi2))

    lo2, hi2 = lax.fori_loop(0, 15, bs_idx,
                             (jnp.int32(0), jnp.int32(Rr * 256)))
    keep_ref[...] = ((s > vk) | (ties & (idx < hi2))).astype(jnp.float32)


def _tc_pool(kk, score):
    return pl.pallas_call(
        functools.partial(_tc_pool_body, kk),
        out_shape=jax.ShapeDtypeStruct((R // 256, 256), jnp.float32),
    )(score.reshape(R // 256, 256))


def _tc_xnext_body(hs_ref, keep_ref, xn_ref):
    xn_ref[...] = hs_ref[...] * keep_ref[...]


def _tc_xnext(hs, keep):
    return pl.pallas_call(
        _tc_xnext_body,
        out_shape=jax.ShapeDtypeStruct((R, 128), jnp.float32),
    )(hs, keep)


def _tc_readout_body(D, hs_ref, keep_ref, brow_ref, bcol_ref, out_ref):
    keep = keep_ref[...]
    xc = hs_ref[:, :D] * keep
    brow = brow_ref[...]
    bcol = bcol_ref[...]
    onehot = ((lax.broadcasted_iota(jnp.int32, (G, R), 0)
               == jnp.broadcast_to(brow, (G, R))).astype(jnp.float32)
              * jnp.broadcast_to(keep.reshape(1, R), (G, R)))
    seg = jnp.dot(onehot, xc, preferred_element_type=jnp.float32)
    cnt = jnp.sum(onehot, axis=1, keepdims=True)
    out_ref[:, D:] = seg / jnp.maximum(cnt, 1.0)
    kcol = keep > 0

    def max_row(g):
        mrow = jnp.max(jnp.where((bcol == g) & kcol, xc, -3e38), axis=0,
                       keepdims=True)
        return jnp.maximum(mrow, 0.0)

    for gb in range(G // 8):
        out_ref[8 * gb:8 * gb + 8, :D] = jnp.concatenate(
            [max_row(8 * gb + j) for j in range(8)], axis=0)


def _tc_readout(D, hs, keep, brow, bcol):
    return pl.pallas_call(
        functools.partial(_tc_readout_body, D),
        out_shape=jax.ShapeDtypeStruct((G, 2 * D), jnp.float32),
    )(hs, keep, brow, bcol)


def _tc_dec_body(x1_ref, x2_ref, x3_ref,
                 w31, b31, g31, be31, w32, b32, g32, be32,
                 w21, b21, g21, be21, w22, b22, g22, be22,
                 w1, bb1, out_ref):
    def bnrelu(h, gam, bet):
        m = jnp.mean(h, axis=0, keepdims=True)
        v = jnp.mean(h * h, axis=0, keepdims=True) - m * m
        h = (h - m) * lax.rsqrt(v + 1e-5) * gam[...] + bet[...]
        return jnp.maximum(h, 0.0)

    def mlp(x, wA, bA, gA, beA, wB, bB, gB, beB):
        h = bnrelu(jnp.dot(x, wA[...], preferred_element_type=jnp.float32)
                   + bA[...], gA, beA)
        return bnrelu(jnp.dot(h, wB[...], preferred_element_type=jnp.float32)
                      + bB[...], gB, beB)

    xd3 = mlp(x3_ref[...], w31, b31, g31, be31, w32, b32, g32, be32)
    xd2 = mlp(xd3 + x2_ref[...], w21, b21, g21, be21, w22, b22, g22, be22)
    logits = (jnp.dot(xd2 + x1_ref[...], w1[...],
                      preferred_element_type=jnp.float32) + bb1[...])
    mx = jnp.max(logits, axis=1, keepdims=True)
    lse = jnp.log(jnp.sum(jnp.exp(logits - mx), axis=1, keepdims=True)) + mx
    out_ref[...] = logits - lse


def _tc_decoder(x1, x2, x3, d3, d2, d1):
    def flat(p):
        return (p["lin1"]["W"], p["lin1"]["b"].reshape(1, -1),
                p["bn1"]["gamma"].reshape(1, -1),
                p["bn1"]["beta"].reshape(1, -1),
                p["lin2"]["W"], p["lin2"]["b"].reshape(1, -1),
                p["bn2"]["gamma"].reshape(1, -1),
                p["bn2"]["beta"].reshape(1, -1))

    return pl.pallas_call(
        _tc_dec_body,
        out_shape=jax.ShapeDtypeStruct((G, d1["W"].shape[1]), jnp.float32),
    )(x1, x2, x3, *flat(d3), *flat(d2), d1["W"], d1["b"].reshape(1, -1))


def _layer(cfg, x, src3, dst3, keep, brow, bcol, conv_p, pool_p):
    agg0, agg1 = _sc_msg(128, x, src3, dst3)
    hs, score = _tc_conv(cfg["N"], x, agg0, agg1, keep, conv_p, pool_p)
    keep_new = _tc_pool(cfg["K"], score).reshape(R, 1)
    xnext = _tc_xnext(hs, keep_new)
    xr = _tc_readout(cfg["H"], hs, keep_new, brow, bcol)
    return xnext, keep_new, xr


def kernel(x, edge_index, batch, params):
    n = x.shape[0]
    xp = jnp.zeros((R, x.shape[1]), jnp.float32).at[:n].set(x)
    n_e = edge_index.shape[1]
    src3 = (jnp.zeros((E_PAD,), jnp.int32).at[:n_e].set(edge_index[0])
            .reshape(NW, CB_E, CH_E))
    dst3 = (jnp.full((E_PAD,), R - 1, jnp.int32).at[:n_e].set(edge_index[1])
            .reshape(NW, CB_E, CH_E))
    bp = jnp.full((R,), -1, jnp.int32).at[:n].set(batch)
    brow = bp.reshape(1, R)
    bcol = bp.reshape(R, 1)
    keep = (jnp.arange(R, dtype=jnp.int32) < n).astype(jnp.float32)
    keep = keep.reshape(R, 1)

    h, keep, x1 = _layer(LAYERS[0], xp, src3, dst3, keep, brow, bcol,
                         params["conv1"], params["pool1"])
    h, keep, x2 = _layer(LAYERS[1], h, src3, dst3, keep, brow, bcol,
                         params["conv2"], params["pool2"])
    h, keep, x3 = _layer(LAYERS[2], h, src3, dst3, keep, brow, bcol,
                         params["conv3"], params["pool3"])
    return _tc_decoder(x1, x2, x3, params["dec3"], params["dec2"],
                       params["dec1"])
